# Initial kernel scaffold; baseline (speedup 1.0000x reference)
import jax
import jax.numpy as jnp
from jax.experimental import pallas as pl


def kernel(center, W1, g1, be1, W2, b2, g2, be2, W3, b3):
    raise NotImplementedError("write your pallas kernel here")


# speedup vs baseline: 371.7499x; 371.7499x over previous
import jax
import jax.numpy as jnp
from jax.experimental import pallas as pl

K_NB = 8
C = 12
R = 256
NBLK = 8


def _jacobi_eigh(a00, a11, a22, a01, a02, a12):
    one = jnp.float32(1.0)
    zero = jnp.float32(0.0)
    A = {(0, 0): a00, (1, 1): a11, (2, 2): a22,
         (0, 1): a01, (0, 2): a02, (1, 2): a12}
    V = {}
    for i in range(3):
        for j in range(3):
            V[(i, j)] = jnp.full_like(a00, one if i == j else zero)

    def getA(i, j):
        return A[(i, j)] if i <= j else A[(j, i)]

    def setA(i, j, v):
        A[(i, j) if i <= j else (j, i)] = v

    for _ in range(15):
        for (p, q) in ((0, 2), (1, 2), (0, 1)):
            r = 3 - p - q
            apq = getA(p, q)
            app = getA(p, p)
            aqq = getA(q, q)
            tau = (aqq - app) / (2.0 * apq)
            t = jnp.sign(tau) / (jnp.abs(tau) + jnp.sqrt(1.0 + tau * tau))
            t = jnp.where(tau == 0.0, one, t)
            small = jnp.abs(apq) <= 1e-6 * jnp.sqrt(jnp.abs(app * aqq))
            t = jnp.where(small | (apq == 0.0), zero, t)
            c = 1.0 / jnp.sqrt(1.0 + t * t)
            s = t * c
            apr = getA(p, r)
            aqr = getA(q, r)
            b_pp = c * app - s * apq
            b_pq = s * app + c * apq
            b_qp = c * apq - s * aqq
            b_qq = s * apq + c * aqq
            b_rp = c * apr - s * aqr
            b_rq = s * apr + c * aqr
            setA(p, p, c * b_pp - s * b_qp)
            setA(q, q, s * b_pq + c * b_qq)
            setA(p, q, c * b_pq - s * b_qq)
            setA(p, r, b_rp)
            setA(q, r, b_rq)
            for i in range(3):
                vip = V[(i, p)]
                viq = V[(i, q)]
                V[(i, p)] = c * vip - s * viq
                V[(i, q)] = s * vip + c * viq

    lam = [A[(0, 0)], A[(1, 1)], A[(2, 2)]]
    cols = [[V[(i, 0)] for i in range(3)],
            [V[(i, 1)] for i in range(3)],
            [V[(i, 2)] for i in range(3)]]

    def cswap(cond, x, y):
        return jnp.where(cond, y, x), jnp.where(cond, x, y)

    for (i, j) in ((0, 1), (1, 2), (0, 1)):
        sw = lam[j] < lam[i]
        lam[i], lam[j] = cswap(sw, lam[i], lam[j])
        for d in range(3):
            cols[i][d], cols[j][d] = cswap(sw, cols[i][d], cols[j][d])
    return lam, cols


def _k1(cb_ref, ca_ref, w1_ref, h1_ref, s1_ref, q1_ref):
    cb = cb_ref[0]
    ca = ca_ref[0]
    N = ca.shape[0]
    cax = ca[:, 0][None, :]
    cay = ca[:, 1][None, :]
    caz = ca[:, 2][None, :]
    d2b = cb[:, 0] ** 2 + cb[:, 1] ** 2 + cb[:, 2] ** 2
    d2a = ca[:, 0] ** 2 + ca[:, 1] ** 2 + ca[:, 2] ** 2
    prod = jnp.dot(cb, ca.T, preferred_element_type=jnp.float32)
    d = d2b[:, None] + d2a[None, :] - 2.0 * prod

    iota = jax.lax.broadcasted_iota(jnp.int32, (R, N), 1)
    gx, gy, gz = [], [], []
    for _ in range(K_NB):
        m = jnp.min(d, axis=1)
        cand = jnp.where(d == m[:, None], iota, N)
        idx = jnp.min(cand, axis=1)
        oh = iota == idx[:, None]
        gx.append(jnp.sum(jnp.where(oh, cax, 0.0), axis=1))
        gy.append(jnp.sum(jnp.where(oh, cay, 0.0), axis=1))
        gz.append(jnp.sum(jnp.where(oh, caz, 0.0), axis=1))
        d = jnp.where(oh, jnp.inf, d)

    gx = jnp.stack(gx, axis=0)
    gy = jnp.stack(gy, axis=0)
    gz = jnp.stack(gz, axis=0)

    def bf(x):
        return x.astype(jnp.bfloat16).astype(jnp.float32)

    def cov_entry(u, v):
        p = [u[j] * v[j] for j in range(K_NB)]
        return ((p[0] + p[1]) + (p[2] + p[3])) + ((p[4] + p[5]) + (p[6] + p[7]))

    dxs = [bf(gx[j:j + 1, :] - gx) for j in range(K_NB)]
    dys = [bf(gy[j:j + 1, :] - gy) for j in range(K_NB)]
    dzs = [bf(gz[j:j + 1, :] - gz) for j in range(K_NB)]
    c00 = cov_entry(dxs, dxs)
    c11 = cov_entry(dys, dys)
    c22 = cov_entry(dzs, dzs)
    c01 = cov_entry(dxs, dys)
    c02 = cov_entry(dxs, dzs)
    c12 = cov_entry(dys, dzs)

    lam, cols = _jacobi_eigh(c00, c11, c22, c01, c02, c12)
    l3, l2, l1 = lam[0], lam[1], lam[2]
    v1 = cols[2]
    v2 = cols[1]
    v3 = [-cols[0][i] for i in range(3)]
    n1 = jnp.sqrt(v1[0] ** 2 + v1[1] ** 2 + v1[2] ** 2)
    n2 = jnp.sqrt(v2[0] ** 2 + v2[1] ** 2 + v2[2] ** 2)
    n3 = jnp.sqrt(v3[0] ** 2 + v3[1] ** 2 + v3[2] ** 2)
    l1 = l1 / n1
    l2 = l2 / n2
    l3 = l3 / n3
    v1 = [v1[i] / n1 for i in range(3)]
    v2 = [v2[i] / n2 for i in range(3)]
    v3 = [v3[i] / n3 for i in range(3)]
    fa = (l1 - l2) / l1
    fp = (l2 - l3) / l1
    fs = l3 / l1
    feats = [fa, fp, fs, v1[0], v1[1], v1[2], v2[0], v2[1], v2[2],
             v3[0], v3[1], v3[2]]

    W1 = w1_ref[...]
    for o in range(C):
        acc = W1[o:o + 1, 0:1] * feats[0]
        for cc in range(1, C):
            acc = acc + W1[o:o + 1, cc:cc + 1] * feats[cc]
        h1_ref[0, 0, o] = acc
        s1_ref[0, 0, o] = jnp.sum(acc, axis=0)
        q1_ref[0, 0, o] = jnp.sum(acc * acc, axis=0)


def _k3(h1_ref, a1_ref, c1_ref, w2_ref, b2_ref, h2_ref, s2_ref, q2_ref):
    a1 = a1_ref[...]
    c1 = c1_ref[...]
    W2 = w2_ref[...]
    b2 = b2_ref[...]
    xs = []
    for cc in range(C):
        xs.append(jax.nn.relu(a1[0:1, cc:cc + 1] * h1_ref[0, 0, cc]
                              + c1[0:1, cc:cc + 1]))
    for o in range(C):
        acc = W2[o:o + 1, 0:1] * xs[0]
        for cc in range(1, C):
            acc = acc + W2[o:o + 1, cc:cc + 1] * xs[cc]
        acc = acc + b2[0:1, o:o + 1]
        h2_ref[0, 0, o] = acc
        s2_ref[0, 0, o] = jnp.sum(acc, axis=0)
        q2_ref[0, 0, o] = jnp.sum(acc * acc, axis=0)


def _k4(h2_ref, a2_ref, c2_ref, w3_ref, b3_ref, o_ref):
    a2 = a2_ref[...]
    c2 = c2_ref[...]
    W3 = w3_ref[...]
    b3 = b3_ref[...]
    xs = []
    for cc in range(C):
        xs.append(jax.nn.relu(a2[0:1, cc:cc + 1] * h2_ref[0, 0, cc]
                              + c2[0:1, cc:cc + 1]))
    for o in range(C):
        acc = W3[o:o + 1, 0:1] * xs[0]
        for cc in range(1, C):
            acc = acc + W3[o:o + 1, cc:cc + 1] * xs[cc]
        acc = acc + b3[0:1, o:o + 1]
        o_ref[0, 0, o] = jnp.sum(acc, axis=0)


def kernel(center, W1, g1, be1, W2, b2, g2, be2, W3, b3):
    B, N, _ = center.shape
    cnt = jnp.float32(B * N * K_NB)

    h1, s1, q1 = pl.pallas_call(
        _k1,
        grid=(B, NBLK),
        in_specs=[
            pl.BlockSpec((1, R, 3), lambda b, i: (b, i, 0)),
            pl.BlockSpec((1, N, 3), lambda b, i: (b, 0, 0)),
            pl.BlockSpec((C, C), lambda b, i: (0, 0)),
        ],
        out_specs=[
            pl.BlockSpec((1, 1, C, K_NB, R), lambda b, i: (b, i, 0, 0, 0)),
            pl.BlockSpec((1, 1, C, R), lambda b, i: (b, i, 0, 0)),
            pl.BlockSpec((1, 1, C, R), lambda b, i: (b, i, 0, 0)),
        ],
        out_shape=[
            jax.ShapeDtypeStruct((B, NBLK, C, K_NB, R), jnp.float32),
            jax.ShapeDtypeStruct((B, NBLK, C, R), jnp.float32),
            jax.ShapeDtypeStruct((B, NBLK, C, R), jnp.float32),
        ],
    )(center, center, W1)

    mean1 = jnp.sum(s1, axis=(0, 1, 3)) / cnt
    var1 = jnp.sum(q1, axis=(0, 1, 3)) / cnt - mean1 * mean1
    a1 = g1 / jnp.sqrt(var1 + 1e-5)
    c1 = be1 - mean1 * a1

    h2, s2, q2 = pl.pallas_call(
        _k3,
        grid=(B, NBLK),
        in_specs=[
            pl.BlockSpec((1, 1, C, K_NB, R), lambda b, i: (b, i, 0, 0, 0)),
            pl.BlockSpec((1, C), lambda b, i: (0, 0)),
            pl.BlockSpec((1, C), lambda b, i: (0, 0)),
            pl.BlockSpec((C, C), lambda b, i: (0, 0)),
            pl.BlockSpec((1, C), lambda b, i: (0, 0)),
        ],
        out_specs=[
            pl.BlockSpec((1, 1, C, K_NB, R), lambda b, i: (b, i, 0, 0, 0)),
            pl.BlockSpec((1, 1, C, R), lambda b, i: (b, i, 0, 0)),
            pl.BlockSpec((1, 1, C, R), lambda b, i: (b, i, 0, 0)),
        ],
        out_shape=[
            jax.ShapeDtypeStruct((B, NBLK, C, K_NB, R), jnp.float32),
            jax.ShapeDtypeStruct((B, NBLK, C, R), jnp.float32),
            jax.ShapeDtypeStruct((B, NBLK, C, R), jnp.float32),
        ],
    )(h1, a1[None, :], c1[None, :], W2, b2[None, :])

    mean2 = jnp.sum(s2, axis=(0, 1, 3)) / cnt
    var2 = jnp.sum(q2, axis=(0, 1, 3)) / cnt - mean2 * mean2
    a2 = g2 / jnp.sqrt(var2 + 1e-5)
    c2 = be2 - mean2 * a2

    out = pl.pallas_call(
        _k4,
        grid=(B, NBLK),
        in_specs=[
            pl.BlockSpec((1, 1, C, K_NB, R), lambda b, i: (b, i, 0, 0, 0)),
            pl.BlockSpec((1, C), lambda b, i: (0, 0)),
            pl.BlockSpec((1, C), lambda b, i: (0, 0)),
            pl.BlockSpec((C, C), lambda b, i: (0, 0)),
            pl.BlockSpec((1, C), lambda b, i: (0, 0)),
        ],
        out_specs=[
            pl.BlockSpec((1, 1, C, R), lambda b, i: (b, i, 0, 0)),
        ],
        out_shape=[
            jax.ShapeDtypeStruct((B, NBLK, C, R), jnp.float32),
        ],
    )(h2, a2[None, :], c2[None, :], W3, b3[None, :])[0]

    return jnp.transpose(out, (0, 2, 1, 3)).reshape(B, C, N)


# speedup vs baseline: 494.4455x; 1.3300x over previous
import jax
import jax.numpy as jnp
from jax.experimental import pallas as pl

K_NB = 8
C = 12
R = 256
NBLK = 8


def _jacobi_eigh(a00, a11, a22, a01, a02, a12):
    one = jnp.float32(1.0)
    zero = jnp.float32(0.0)
    A = {(0, 0): a00, (1, 1): a11, (2, 2): a22,
         (0, 1): a01, (0, 2): a02, (1, 2): a12}
    V = {}
    for i in range(3):
        for j in range(3):
            V[(i, j)] = jnp.full_like(a00, one if i == j else zero)

    def getA(i, j):
        return A[(i, j)] if i <= j else A[(j, i)]

    def setA(i, j, v):
        A[(i, j) if i <= j else (j, i)] = v

    for _ in range(15):
        for (p, q) in ((0, 2), (1, 2), (0, 1)):
            r = 3 - p - q
            apq = getA(p, q)
            app = getA(p, p)
            aqq = getA(q, q)
            tau = (aqq - app) / (2.0 * apq)
            t = jnp.sign(tau) / (jnp.abs(tau) + jnp.sqrt(1.0 + tau * tau))
            t = jnp.where(tau == 0.0, one, t)
            small = jnp.abs(apq) <= 1e-6 * jnp.sqrt(jnp.abs(app * aqq))
            t = jnp.where(small | (apq == 0.0), zero, t)
            c = 1.0 / jnp.sqrt(1.0 + t * t)
            s = t * c
            apr = getA(p, r)
            aqr = getA(q, r)
            b_pp = c * app - s * apq
            b_pq = s * app + c * apq
            b_qp = c * apq - s * aqq
            b_qq = s * apq + c * aqq
            b_rp = c * apr - s * aqr
            b_rq = s * apr + c * aqr
            setA(p, p, c * b_pp - s * b_qp)
            setA(q, q, s * b_pq + c * b_qq)
            setA(p, q, c * b_pq - s * b_qq)
            setA(p, r, b_rp)
            setA(q, r, b_rq)
            for i in range(3):
                vip = V[(i, p)]
                viq = V[(i, q)]
                V[(i, p)] = c * vip - s * viq
                V[(i, q)] = s * vip + c * viq

    lam = [A[(0, 0)], A[(1, 1)], A[(2, 2)]]
    cols = [[V[(i, 0)] for i in range(3)],
            [V[(i, 1)] for i in range(3)],
            [V[(i, 2)] for i in range(3)]]

    def cswap(cond, x, y):
        return jnp.where(cond, y, x), jnp.where(cond, x, y)

    for (i, j) in ((0, 1), (1, 2), (0, 1)):
        sw = lam[j] < lam[i]
        lam[i], lam[j] = cswap(sw, lam[i], lam[j])
        for d in range(3):
            cols[i][d], cols[j][d] = cswap(sw, cols[i][d], cols[j][d])
    return lam, cols


def _k1(cb_ref, ca_ref, w1_ref, h1_ref, s1_ref, q1_ref):
    cb = cb_ref[0]
    ca = ca_ref[0]
    N = ca.shape[0]
    cax = ca[:, 0][None, :]
    cay = ca[:, 1][None, :]
    caz = ca[:, 2][None, :]
    d2b = cb[:, 0] ** 2 + cb[:, 1] ** 2 + cb[:, 2] ** 2
    d2a = ca[:, 0] ** 2 + ca[:, 1] ** 2 + ca[:, 2] ** 2
    prod = jnp.dot(cb, ca.T, preferred_element_type=jnp.float32)
    d = d2b[:, None] + d2a[None, :] - 2.0 * prod

    iota = jax.lax.broadcasted_iota(jnp.int32, (R, N), 1)
    idxs = []
    for _ in range(K_NB):
        m = jnp.min(d, axis=1)
        cand = jnp.where(d == m[:, None], iota, N)
        idx = jnp.min(cand, axis=1)
        idxs.append(idx)
        d = jnp.where(iota == idx[:, None], jnp.inf, d)

    idx8 = jnp.stack(idxs, axis=0)
    ch = idx8 // 128
    li = idx8 - ch * 128

    def gather_row(vec_row):
        out = jnp.zeros((K_NB, R), jnp.float32)
        for k in range(N // 128):
            src = jnp.broadcast_to(vec_row[:, k * 128:(k + 1) * 128],
                                   (K_NB, 128))
            val = jnp.take_along_axis(src, li, axis=1)
            out = jnp.where(ch == k, val, out)
        return out

    gx = gather_row(cax)
    gy = gather_row(cay)
    gz = gather_row(caz)

    def bf(x):
        return x.astype(jnp.bfloat16).astype(jnp.float32)

    def cov_entry(u, v):
        p = [u[j] * v[j] for j in range(K_NB)]
        return ((p[0] + p[1]) + (p[2] + p[3])) + ((p[4] + p[5]) + (p[6] + p[7]))

    dxs = [bf(gx[j:j + 1, :] - gx) for j in range(K_NB)]
    dys = [bf(gy[j:j + 1, :] - gy) for j in range(K_NB)]
    dzs = [bf(gz[j:j + 1, :] - gz) for j in range(K_NB)]
    c00 = cov_entry(dxs, dxs)
    c11 = cov_entry(dys, dys)
    c22 = cov_entry(dzs, dzs)
    c01 = cov_entry(dxs, dys)
    c02 = cov_entry(dxs, dzs)
    c12 = cov_entry(dys, dzs)

    lam, cols = _jacobi_eigh(c00, c11, c22, c01, c02, c12)
    l3, l2, l1 = lam[0], lam[1], lam[2]
    v1 = cols[2]
    v2 = cols[1]
    v3 = [-cols[0][i] for i in range(3)]
    n1 = jnp.sqrt(v1[0] ** 2 + v1[1] ** 2 + v1[2] ** 2)
    n2 = jnp.sqrt(v2[0] ** 2 + v2[1] ** 2 + v2[2] ** 2)
    n3 = jnp.sqrt(v3[0] ** 2 + v3[1] ** 2 + v3[2] ** 2)
    l1 = l1 / n1
    l2 = l2 / n2
    l3 = l3 / n3
    v1 = [v1[i] / n1 for i in range(3)]
    v2 = [v2[i] / n2 for i in range(3)]
    v3 = [v3[i] / n3 for i in range(3)]
    fa = (l1 - l2) / l1
    fp = (l2 - l3) / l1
    fs = l3 / l1
    feats = [fa, fp, fs, v1[0], v1[1], v1[2], v2[0], v2[1], v2[2],
             v3[0], v3[1], v3[2]]

    W1 = w1_ref[...]
    for o in range(C):
        acc = W1[o:o + 1, 0:1] * feats[0]
        for cc in range(1, C):
            acc = acc + W1[o:o + 1, cc:cc + 1] * feats[cc]
        h1_ref[0, 0, o] = acc
        s1_ref[0, 0, o] = jnp.sum(acc, axis=0)
        q1_ref[0, 0, o] = jnp.sum(acc * acc, axis=0)


def _k3(h1_ref, a1_ref, c1_ref, w2_ref, b2_ref, h2_ref, s2_ref, q2_ref):
    a1 = a1_ref[...]
    c1 = c1_ref[...]
    W2 = w2_ref[...]
    b2 = b2_ref[...]
    xs = []
    for cc in range(C):
        xs.append(jax.nn.relu(a1[0:1, cc:cc + 1] * h1_ref[0, 0, cc]
                              + c1[0:1, cc:cc + 1]))
    for o in range(C):
        acc = W2[o:o + 1, 0:1] * xs[0]
        for cc in range(1, C):
            acc = acc + W2[o:o + 1, cc:cc + 1] * xs[cc]
        acc = acc + b2[0:1, o:o + 1]
        h2_ref[0, 0, o] = acc
        s2_ref[0, 0, o] = jnp.sum(acc, axis=0)
        q2_ref[0, 0, o] = jnp.sum(acc * acc, axis=0)


def _k4(h2_ref, a2_ref, c2_ref, w3_ref, b3_ref, o_ref):
    a2 = a2_ref[...]
    c2 = c2_ref[...]
    W3 = w3_ref[...]
    b3 = b3_ref[...]
    xs = []
    for cc in range(C):
        xs.append(jax.nn.relu(a2[0:1, cc:cc + 1] * h2_ref[0, 0, cc]
                              + c2[0:1, cc:cc + 1]))
    for o in range(C):
        acc = W3[o:o + 1, 0:1] * xs[0]
        for cc in range(1, C):
            acc = acc + W3[o:o + 1, cc:cc + 1] * xs[cc]
        acc = acc + b3[0:1, o:o + 1]
        o_ref[0, 0, o] = jnp.sum(acc, axis=0)


def kernel(center, W1, g1, be1, W2, b2, g2, be2, W3, b3):
    B, N, _ = center.shape
    cnt = jnp.float32(B * N * K_NB)

    h1, s1, q1 = pl.pallas_call(
        _k1,
        grid=(B, NBLK),
        in_specs=[
            pl.BlockSpec((1, R, 3), lambda b, i: (b, i, 0)),
            pl.BlockSpec((1, N, 3), lambda b, i: (b, 0, 0)),
            pl.BlockSpec((C, C), lambda b, i: (0, 0)),
        ],
        out_specs=[
            pl.BlockSpec((1, 1, C, K_NB, R), lambda b, i: (b, i, 0, 0, 0)),
            pl.BlockSpec((1, 1, C, R), lambda b, i: (b, i, 0, 0)),
            pl.BlockSpec((1, 1, C, R), lambda b, i: (b, i, 0, 0)),
        ],
        out_shape=[
            jax.ShapeDtypeStruct((B, NBLK, C, K_NB, R), jnp.float32),
            jax.ShapeDtypeStruct((B, NBLK, C, R), jnp.float32),
            jax.ShapeDtypeStruct((B, NBLK, C, R), jnp.float32),
        ],
    )(center, center, W1)

    mean1 = jnp.sum(s1, axis=(0, 1, 3)) / cnt
    var1 = jnp.sum(q1, axis=(0, 1, 3)) / cnt - mean1 * mean1
    a1 = g1 / jnp.sqrt(var1 + 1e-5)
    c1 = be1 - mean1 * a1

    h2, s2, q2 = pl.pallas_call(
        _k3,
        grid=(B, NBLK),
        in_specs=[
            pl.BlockSpec((1, 1, C, K_NB, R), lambda b, i: (b, i, 0, 0, 0)),
            pl.BlockSpec((1, C), lambda b, i: (0, 0)),
            pl.BlockSpec((1, C), lambda b, i: (0, 0)),
            pl.BlockSpec((C, C), lambda b, i: (0, 0)),
            pl.BlockSpec((1, C), lambda b, i: (0, 0)),
        ],
        out_specs=[
            pl.BlockSpec((1, 1, C, K_NB, R), lambda b, i: (b, i, 0, 0, 0)),
            pl.BlockSpec((1, 1, C, R), lambda b, i: (b, i, 0, 0)),
            pl.BlockSpec((1, 1, C, R), lambda b, i: (b, i, 0, 0)),
        ],
        out_shape=[
            jax.ShapeDtypeStruct((B, NBLK, C, K_NB, R), jnp.float32),
            jax.ShapeDtypeStruct((B, NBLK, C, R), jnp.float32),
            jax.ShapeDtypeStruct((B, NBLK, C, R), jnp.float32),
        ],
    )(h1, a1[None, :], c1[None, :], W2, b2[None, :])

    mean2 = jnp.sum(s2, axis=(0, 1, 3)) / cnt
    var2 = jnp.sum(q2, axis=(0, 1, 3)) / cnt - mean2 * mean2
    a2 = g2 / jnp.sqrt(var2 + 1e-5)
    c2 = be2 - mean2 * a2

    out = pl.pallas_call(
        _k4,
        grid=(B, NBLK),
        in_specs=[
            pl.BlockSpec((1, 1, C, K_NB, R), lambda b, i: (b, i, 0, 0, 0)),
            pl.BlockSpec((1, C), lambda b, i: (0, 0)),
            pl.BlockSpec((1, C), lambda b, i: (0, 0)),
            pl.BlockSpec((C, C), lambda b, i: (0, 0)),
            pl.BlockSpec((1, C), lambda b, i: (0, 0)),
        ],
        out_specs=[
            pl.BlockSpec((1, 1, C, R), lambda b, i: (b, i, 0, 0)),
        ],
        out_shape=[
            jax.ShapeDtypeStruct((B, NBLK, C, R), jnp.float32),
        ],
    )(h2, a2[None, :], c2[None, :], W3, b3[None, :])[0]

    return jnp.transpose(out, (0, 2, 1, 3)).reshape(B, C, N)


# speedup vs baseline: 549.2899x; 1.1109x over previous
import jax
import jax.numpy as jnp
from jax.experimental import pallas as pl

K_NB = 8
C = 12
R = 256
NBLK = 8


def _jacobi_eigh(a00, a11, a22, a01, a02, a12):
    one = jnp.float32(1.0)
    zero = jnp.float32(0.0)
    A = {(0, 0): a00, (1, 1): a11, (2, 2): a22,
         (0, 1): a01, (0, 2): a02, (1, 2): a12}
    V = {}
    for i in range(3):
        for j in range(3):
            V[(i, j)] = jnp.full_like(a00, one if i == j else zero)

    def getA(i, j):
        return A[(i, j)] if i <= j else A[(j, i)]

    def setA(i, j, v):
        A[(i, j) if i <= j else (j, i)] = v

    for _ in range(15):
        for (p, q) in ((0, 2), (1, 2), (0, 1)):
            r = 3 - p - q
            apq = getA(p, q)
            app = getA(p, p)
            aqq = getA(q, q)
            tau = (aqq - app) / (2.0 * apq)
            t = jnp.sign(tau) / (jnp.abs(tau) + jnp.sqrt(1.0 + tau * tau))
            t = jnp.where(tau == 0.0, one, t)
            small = jnp.abs(apq) <= 1e-6 * jnp.sqrt(jnp.abs(app * aqq))
            t = jnp.where(small | (apq == 0.0), zero, t)
            c = 1.0 / jnp.sqrt(1.0 + t * t)
            s = t * c
            apr = getA(p, r)
            aqr = getA(q, r)
            b_pp = c * app - s * apq
            b_pq = s * app + c * apq
            b_qp = c * apq - s * aqq
            b_qq = s * apq + c * aqq
            b_rp = c * apr - s * aqr
            b_rq = s * apr + c * aqr
            setA(p, p, c * b_pp - s * b_qp)
            setA(q, q, s * b_pq + c * b_qq)
            setA(p, q, c * b_pq - s * b_qq)
            setA(p, r, b_rp)
            setA(q, r, b_rq)
            for i in range(3):
                vip = V[(i, p)]
                viq = V[(i, q)]
                V[(i, p)] = c * vip - s * viq
                V[(i, q)] = s * vip + c * viq

    lam = [A[(0, 0)], A[(1, 1)], A[(2, 2)]]
    cols = [[V[(i, 0)] for i in range(3)],
            [V[(i, 1)] for i in range(3)],
            [V[(i, 2)] for i in range(3)]]

    def cswap(cond, x, y):
        return jnp.where(cond, y, x), jnp.where(cond, x, y)

    for (i, j) in ((0, 1), (1, 2), (0, 1)):
        sw = lam[j] < lam[i]
        lam[i], lam[j] = cswap(sw, lam[i], lam[j])
        for d in range(3):
            cols[i][d], cols[j][d] = cswap(sw, cols[i][d], cols[j][d])
    return lam, cols


def _k1(cb_ref, cat_ref, w1_ref, h1_ref, s1_ref, q1_ref):
    cb = cb_ref[0]
    caT = cat_ref[0]
    N = caT.shape[1]
    cax = caT[0:1, :]
    cay = caT[1:2, :]
    caz = caT[2:3, :]
    d2b = cb[:, 0] ** 2 + cb[:, 1] ** 2 + cb[:, 2] ** 2
    d2a = cax * cax + cay * cay + caz * caz
    prod = jnp.dot(cb, caT, preferred_element_type=jnp.float32)
    d = d2b[:, None] + d2a - 2.0 * prod

    iota = jax.lax.broadcasted_iota(jnp.int32, (R, N), 1)
    idxs = []
    for _ in range(K_NB):
        m = jnp.min(d, axis=1)
        cand = jnp.where(d == m[:, None], iota, N)
        idx = jnp.min(cand, axis=1)
        idxs.append(idx)
        d = jnp.where(iota == idx[:, None], jnp.inf, d)

    idx8 = jnp.stack(idxs, axis=0)
    ch = idx8 // 128
    li = idx8 - ch * 128

    def gather_row(vec_row):
        out = jnp.zeros((K_NB, R), jnp.float32)
        for k in range(N // 128):
            src = jnp.broadcast_to(vec_row[:, k * 128:(k + 1) * 128],
                                   (K_NB, 128))
            val = jnp.take_along_axis(src, li, axis=1)
            out = jnp.where(ch == k, val, out)
        return out

    gx = gather_row(cax)
    gy = gather_row(cay)
    gz = gather_row(caz)

    def bf(x):
        return x.astype(jnp.bfloat16).astype(jnp.float32)

    def cov_entry(u, v):
        p = [u[j] * v[j] for j in range(K_NB)]
        return ((p[0] + p[1]) + (p[2] + p[3])) + ((p[4] + p[5]) + (p[6] + p[7]))

    dxs = [bf(gx[j:j + 1, :] - gx) for j in range(K_NB)]
    dys = [bf(gy[j:j + 1, :] - gy) for j in range(K_NB)]
    dzs = [bf(gz[j:j + 1, :] - gz) for j in range(K_NB)]
    c00 = cov_entry(dxs, dxs)
    c11 = cov_entry(dys, dys)
    c22 = cov_entry(dzs, dzs)
    c01 = cov_entry(dxs, dys)
    c02 = cov_entry(dxs, dzs)
    c12 = cov_entry(dys, dzs)

    lam, cols = _jacobi_eigh(c00, c11, c22, c01, c02, c12)
    l3, l2, l1 = lam[0], lam[1], lam[2]
    v1 = cols[2]
    v2 = cols[1]
    v3 = [-cols[0][i] for i in range(3)]
    n1 = jnp.sqrt(v1[0] ** 2 + v1[1] ** 2 + v1[2] ** 2)
    n2 = jnp.sqrt(v2[0] ** 2 + v2[1] ** 2 + v2[2] ** 2)
    n3 = jnp.sqrt(v3[0] ** 2 + v3[1] ** 2 + v3[2] ** 2)
    l1 = l1 / n1
    l2 = l2 / n2
    l3 = l3 / n3
    v1 = [v1[i] / n1 for i in range(3)]
    v2 = [v2[i] / n2 for i in range(3)]
    v3 = [v3[i] / n3 for i in range(3)]
    fa = (l1 - l2) / l1
    fp = (l2 - l3) / l1
    fs = l3 / l1
    feats = [fa, fp, fs, v1[0], v1[1], v1[2], v2[0], v2[1], v2[2],
             v3[0], v3[1], v3[2]]

    first = (pl.program_id(0) == 0) & (pl.program_id(1) == 0)

    @pl.when(first)
    def _():
        s1_ref[...] = jnp.zeros_like(s1_ref)
        q1_ref[...] = jnp.zeros_like(q1_ref)

    W1 = w1_ref[...]
    for o in range(C):
        acc = W1[o:o + 1, 0:1] * feats[0]
        for cc in range(1, C):
            acc = acc + W1[o:o + 1, cc:cc + 1] * feats[cc]
        h1_ref[0, 0, o] = acc
        s1_ref[o, :] = s1_ref[o, :] + jnp.sum(acc, axis=0)
        q1_ref[o, :] = q1_ref[o, :] + jnp.sum(acc * acc, axis=0)


def _k3(h1_ref, a1_ref, c1_ref, w2_ref, b2_ref, h2_ref, s2_ref, q2_ref):
    first = (pl.program_id(0) == 0) & (pl.program_id(1) == 0)

    @pl.when(first)
    def _():
        s2_ref[...] = jnp.zeros_like(s2_ref)
        q2_ref[...] = jnp.zeros_like(q2_ref)

    a1 = a1_ref[...]
    c1 = c1_ref[...]
    W2 = w2_ref[...]
    b2 = b2_ref[...]
    xs = []
    for cc in range(C):
        xs.append(jax.nn.relu(a1[0:1, cc:cc + 1] * h1_ref[0, 0, cc]
                              + c1[0:1, cc:cc + 1]))
    for o in range(C):
        acc = W2[o:o + 1, 0:1] * xs[0]
        for cc in range(1, C):
            acc = acc + W2[o:o + 1, cc:cc + 1] * xs[cc]
        acc = acc + b2[0:1, o:o + 1]
        h2_ref[0, 0, o] = acc
        s2_ref[o, :] = s2_ref[o, :] + jnp.sum(acc, axis=0)
        q2_ref[o, :] = q2_ref[o, :] + jnp.sum(acc * acc, axis=0)


def _k4(h2_ref, a2_ref, c2_ref, w3_ref, b3_ref, o_ref):
    a2 = a2_ref[...]
    c2 = c2_ref[...]
    W3 = w3_ref[...]
    b3 = b3_ref[...]
    xs = []
    for cc in range(C):
        xs.append(jax.nn.relu(a2[0:1, cc:cc + 1] * h2_ref[0, 0, cc]
                              + c2[0:1, cc:cc + 1]))
    for o in range(C):
        acc = W3[o:o + 1, 0:1] * xs[0]
        for cc in range(1, C):
            acc = acc + W3[o:o + 1, cc:cc + 1] * xs[cc]
        acc = acc + b3[0:1, o:o + 1]
        o_ref[0, o, 0, 0] = jnp.sum(acc, axis=0)


def kernel(center, W1, g1, be1, W2, b2, g2, be2, W3, b3):
    B, N, _ = center.shape
    cnt = jnp.float32(B * N * K_NB)

    centerT = jnp.transpose(center, (0, 2, 1))
    h1, s1, q1 = pl.pallas_call(
        _k1,
        grid=(B, NBLK),
        in_specs=[
            pl.BlockSpec((1, R, 3), lambda b, i: (b, i, 0)),
            pl.BlockSpec((1, 3, N), lambda b, i: (b, 0, 0)),
            pl.BlockSpec((C, C), lambda b, i: (0, 0)),
        ],
        out_specs=[
            pl.BlockSpec((1, 1, C, K_NB, R), lambda b, i: (b, i, 0, 0, 0)),
            pl.BlockSpec((C, R), lambda b, i: (0, 0)),
            pl.BlockSpec((C, R), lambda b, i: (0, 0)),
        ],
        out_shape=[
            jax.ShapeDtypeStruct((B, NBLK, C, K_NB, R), jnp.float32),
            jax.ShapeDtypeStruct((C, R), jnp.float32),
            jax.ShapeDtypeStruct((C, R), jnp.float32),
        ],
    )(center, centerT, W1)

    mean1 = jnp.sum(s1, axis=1) / cnt
    var1 = jnp.sum(q1, axis=1) / cnt - mean1 * mean1
    a1 = g1 / jnp.sqrt(var1 + 1e-5)
    c1 = be1 - mean1 * a1

    h2, s2, q2 = pl.pallas_call(
        _k3,
        grid=(B, NBLK),
        in_specs=[
            pl.BlockSpec((1, 1, C, K_NB, R), lambda b, i: (b, i, 0, 0, 0)),
            pl.BlockSpec((1, C), lambda b, i: (0, 0)),
            pl.BlockSpec((1, C), lambda b, i: (0, 0)),
            pl.BlockSpec((C, C), lambda b, i: (0, 0)),
            pl.BlockSpec((1, C), lambda b, i: (0, 0)),
        ],
        out_specs=[
            pl.BlockSpec((1, 1, C, K_NB, R), lambda b, i: (b, i, 0, 0, 0)),
            pl.BlockSpec((C, R), lambda b, i: (0, 0)),
            pl.BlockSpec((C, R), lambda b, i: (0, 0)),
        ],
        out_shape=[
            jax.ShapeDtypeStruct((B, NBLK, C, K_NB, R), jnp.float32),
            jax.ShapeDtypeStruct((C, R), jnp.float32),
            jax.ShapeDtypeStruct((C, R), jnp.float32),
        ],
    )(h1, a1[None, :], c1[None, :], W2, b2[None, :])

    mean2 = jnp.sum(s2, axis=1) / cnt
    var2 = jnp.sum(q2, axis=1) / cnt - mean2 * mean2
    a2 = g2 / jnp.sqrt(var2 + 1e-5)
    c2 = be2 - mean2 * a2

    out = pl.pallas_call(
        _k4,
        grid=(B, NBLK),
        in_specs=[
            pl.BlockSpec((1, 1, C, K_NB, R), lambda b, i: (b, i, 0, 0, 0)),
            pl.BlockSpec((1, C), lambda b, i: (0, 0)),
            pl.BlockSpec((1, C), lambda b, i: (0, 0)),
            pl.BlockSpec((C, C), lambda b, i: (0, 0)),
            pl.BlockSpec((1, C), lambda b, i: (0, 0)),
        ],
        out_specs=[
            pl.BlockSpec((1, C, 1, 1, R), lambda b, i: (b, 0, i, 0, 0)),
        ],
        out_shape=[
            jax.ShapeDtypeStruct((B, C, NBLK, 1, R), jnp.float32),
        ],
    )(h2, a2[None, :], c2[None, :], W3, b3[None, :])[0]

    return out.reshape(B, C, N)


# speedup vs baseline: 647.9500x; 1.1796x over previous
import jax
import jax.numpy as jnp
from jax.experimental import pallas as pl

K_NB = 8
C = 12
R = 256
NBLK = 8


def _jacobi_eigh(a00, a11, a22, a01, a02, a12):
    one = jnp.float32(1.0)
    zero = jnp.float32(0.0)
    A = {(0, 0): a00, (1, 1): a11, (2, 2): a22,
         (0, 1): a01, (0, 2): a02, (1, 2): a12}
    V = {}
    for i in range(3):
        for j in range(3):
            V[(i, j)] = jnp.full_like(a00, one if i == j else zero)

    def getA(i, j):
        return A[(i, j)] if i <= j else A[(j, i)]

    def setA(i, j, v):
        A[(i, j) if i <= j else (j, i)] = v

    for _ in range(15):
        for (p, q) in ((0, 2), (1, 2), (0, 1)):
            r = 3 - p - q
            apq = getA(p, q)
            app = getA(p, p)
            aqq = getA(q, q)
            tau = (aqq - app) / (2.0 * apq)
            t = jnp.sign(tau) / (jnp.abs(tau) + jnp.sqrt(1.0 + tau * tau))
            t = jnp.where(tau == 0.0, one, t)
            small = jnp.abs(apq) <= 1e-6 * jnp.sqrt(jnp.abs(app * aqq))
            t = jnp.where(small | (apq == 0.0), zero, t)
            c = 1.0 / jnp.sqrt(1.0 + t * t)
            s = t * c
            apr = getA(p, r)
            aqr = getA(q, r)
            b_pp = c * app - s * apq
            b_pq = s * app + c * apq
            b_qp = c * apq - s * aqq
            b_qq = s * apq + c * aqq
            b_rp = c * apr - s * aqr
            b_rq = s * apr + c * aqr
            setA(p, p, c * b_pp - s * b_qp)
            setA(q, q, s * b_pq + c * b_qq)
            setA(p, q, c * b_pq - s * b_qq)
            setA(p, r, b_rp)
            setA(q, r, b_rq)
            for i in range(3):
                vip = V[(i, p)]
                viq = V[(i, q)]
                V[(i, p)] = c * vip - s * viq
                V[(i, q)] = s * vip + c * viq

    lam = [A[(0, 0)], A[(1, 1)], A[(2, 2)]]
    cols = [[V[(i, 0)] for i in range(3)],
            [V[(i, 1)] for i in range(3)],
            [V[(i, 2)] for i in range(3)]]

    def cswap(cond, x, y):
        return jnp.where(cond, y, x), jnp.where(cond, x, y)

    for (i, j) in ((0, 1), (1, 2), (0, 1)):
        sw = lam[j] < lam[i]
        lam[i], lam[j] = cswap(sw, lam[i], lam[j])
        for d in range(3):
            cols[i][d], cols[j][d] = cswap(sw, cols[i][d], cols[j][d])
    return lam, cols


def _k1(cb_ref, cat_ref, w1_ref, h1_ref, s1_ref, q1_ref):
    cb = cb_ref[0]
    caT = cat_ref[0]
    N = caT.shape[1]
    cax = caT[0:1, :]
    cay = caT[1:2, :]
    caz = caT[2:3, :]
    d2b = cb[:, 0] ** 2 + cb[:, 1] ** 2 + cb[:, 2] ** 2
    d2a = cax * cax + cay * cay + caz * caz
    prod = jnp.dot(cb, caT, preferred_element_type=jnp.float32)
    d = d2b[:, None] + d2a - 2.0 * prod

    iota = jax.lax.broadcasted_iota(jnp.int32, (R, N), 1).astype(jnp.float32)
    idxs = []
    for _ in range(K_NB):
        m = jnp.min(d, axis=1)
        cand = jnp.where(d == m[:, None], iota, jnp.float32(N))
        idx = jnp.min(cand, axis=1)
        idxs.append(idx)
        d = jnp.where(iota == idx[:, None], jnp.inf, d)

    idx8 = jnp.stack(idxs, axis=0).astype(jnp.int32)
    ch = idx8 // 128
    li = idx8 - ch * 128

    def gather_row(vec_row):
        out = jnp.zeros((K_NB, R), jnp.float32)
        for k in range(N // 128):
            src = jnp.broadcast_to(vec_row[:, k * 128:(k + 1) * 128],
                                   (K_NB, 128))
            val = jnp.take_along_axis(src, li, axis=1)
            out = jnp.where(ch == k, val, out)
        return out

    gx = gather_row(cax)
    gy = gather_row(cay)
    gz = gather_row(caz)

    def bf(x):
        return x.astype(jnp.bfloat16).astype(jnp.float32)

    def cov_entry(u, v):
        p = [u[j] * v[j] for j in range(K_NB)]
        return ((p[0] + p[1]) + (p[2] + p[3])) + ((p[4] + p[5]) + (p[6] + p[7]))

    dxs = [bf(gx[j:j + 1, :] - gx) for j in range(K_NB)]
    dys = [bf(gy[j:j + 1, :] - gy) for j in range(K_NB)]
    dzs = [bf(gz[j:j + 1, :] - gz) for j in range(K_NB)]
    c00 = cov_entry(dxs, dxs)
    c11 = cov_entry(dys, dys)
    c22 = cov_entry(dzs, dzs)
    c01 = cov_entry(dxs, dys)
    c02 = cov_entry(dxs, dzs)
    c12 = cov_entry(dys, dzs)

    lam, cols = _jacobi_eigh(c00, c11, c22, c01, c02, c12)
    l3, l2, l1 = lam[0], lam[1], lam[2]
    v1 = cols[2]
    v2 = cols[1]
    v3 = [-cols[0][i] for i in range(3)]
    n1 = jnp.sqrt(v1[0] ** 2 + v1[1] ** 2 + v1[2] ** 2)
    n2 = jnp.sqrt(v2[0] ** 2 + v2[1] ** 2 + v2[2] ** 2)
    n3 = jnp.sqrt(v3[0] ** 2 + v3[1] ** 2 + v3[2] ** 2)
    l1 = l1 / n1
    l2 = l2 / n2
    l3 = l3 / n3
    v1 = [v1[i] / n1 for i in range(3)]
    v2 = [v2[i] / n2 for i in range(3)]
    v3 = [v3[i] / n3 for i in range(3)]
    fa = (l1 - l2) / l1
    fp = (l2 - l3) / l1
    fs = l3 / l1
    feats = [fa, fp, fs, v1[0], v1[1], v1[2], v2[0], v2[1], v2[2],
             v3[0], v3[1], v3[2]]

    first = (pl.program_id(0) == 0) & (pl.program_id(1) == 0)

    @pl.when(first)
    def _():
        s1_ref[...] = jnp.zeros_like(s1_ref)
        q1_ref[...] = jnp.zeros_like(q1_ref)

    W1 = w1_ref[...]
    for o in range(C):
        acc = W1[o:o + 1, 0:1] * feats[0]
        for cc in range(1, C):
            acc = acc + W1[o:o + 1, cc:cc + 1] * feats[cc]
        h1_ref[0, 0, o] = acc
        s1_ref[o, :] = s1_ref[o, :] + jnp.sum(acc, axis=0)
        q1_ref[o, :] = q1_ref[o, :] + jnp.sum(acc * acc, axis=0)


def _k3(h1_ref, a1_ref, c1_ref, w2_ref, b2_ref, h2_ref, s2_ref, q2_ref):
    first = (pl.program_id(0) == 0) & (pl.program_id(1) == 0)

    @pl.when(first)
    def _():
        s2_ref[...] = jnp.zeros_like(s2_ref)
        q2_ref[...] = jnp.zeros_like(q2_ref)

    a1 = a1_ref[...]
    c1 = c1_ref[...]
    W2 = w2_ref[...]
    b2 = b2_ref[...]
    xs = []
    for cc in range(C):
        xs.append(jax.nn.relu(a1[0:1, cc:cc + 1] * h1_ref[0, 0, cc]
                              + c1[0:1, cc:cc + 1]))
    for o in range(C):
        acc = W2[o:o + 1, 0:1] * xs[0]
        for cc in range(1, C):
            acc = acc + W2[o:o + 1, cc:cc + 1] * xs[cc]
        acc = acc + b2[0:1, o:o + 1]
        h2_ref[0, 0, o] = acc
        s2_ref[o, :] = s2_ref[o, :] + jnp.sum(acc, axis=0)
        q2_ref[o, :] = q2_ref[o, :] + jnp.sum(acc * acc, axis=0)


def _k4(h2_ref, a2_ref, c2_ref, w3_ref, b3_ref, o_ref):
    a2 = a2_ref[...]
    c2 = c2_ref[...]
    W3 = w3_ref[...]
    b3 = b3_ref[...]
    xs = []
    for cc in range(C):
        xs.append(jax.nn.relu(a2[0:1, cc:cc + 1] * h2_ref[0, 0, cc]
                              + c2[0:1, cc:cc + 1]))
    for o in range(C):
        acc = W3[o:o + 1, 0:1] * xs[0]
        for cc in range(1, C):
            acc = acc + W3[o:o + 1, cc:cc + 1] * xs[cc]
        acc = acc + b3[0:1, o:o + 1]
        o_ref[0, o, 0, 0] = jnp.sum(acc, axis=0)


def kernel(center, W1, g1, be1, W2, b2, g2, be2, W3, b3):
    B, N, _ = center.shape
    cnt = jnp.float32(B * N * K_NB)

    centerT = jnp.transpose(center, (0, 2, 1))
    h1, s1, q1 = pl.pallas_call(
        _k1,
        grid=(B, NBLK),
        in_specs=[
            pl.BlockSpec((1, R, 3), lambda b, i: (b, i, 0)),
            pl.BlockSpec((1, 3, N), lambda b, i: (b, 0, 0)),
            pl.BlockSpec((C, C), lambda b, i: (0, 0)),
        ],
        out_specs=[
            pl.BlockSpec((1, 1, C, K_NB, R), lambda b, i: (b, i, 0, 0, 0)),
            pl.BlockSpec((C, R), lambda b, i: (0, 0)),
            pl.BlockSpec((C, R), lambda b, i: (0, 0)),
        ],
        out_shape=[
            jax.ShapeDtypeStruct((B, NBLK, C, K_NB, R), jnp.float32),
            jax.ShapeDtypeStruct((C, R), jnp.float32),
            jax.ShapeDtypeStruct((C, R), jnp.float32),
        ],
    )(center, centerT, W1)

    mean1 = jnp.sum(s1, axis=1) / cnt
    var1 = jnp.sum(q1, axis=1) / cnt - mean1 * mean1
    a1 = g1 / jnp.sqrt(var1 + 1e-5)
    c1 = be1 - mean1 * a1

    h2, s2, q2 = pl.pallas_call(
        _k3,
        grid=(B, NBLK),
        in_specs=[
            pl.BlockSpec((1, 1, C, K_NB, R), lambda b, i: (b, i, 0, 0, 0)),
            pl.BlockSpec((1, C), lambda b, i: (0, 0)),
            pl.BlockSpec((1, C), lambda b, i: (0, 0)),
            pl.BlockSpec((C, C), lambda b, i: (0, 0)),
            pl.BlockSpec((1, C), lambda b, i: (0, 0)),
        ],
        out_specs=[
            pl.BlockSpec((1, 1, C, K_NB, R), lambda b, i: (b, i, 0, 0, 0)),
            pl.BlockSpec((C, R), lambda b, i: (0, 0)),
            pl.BlockSpec((C, R), lambda b, i: (0, 0)),
        ],
        out_shape=[
            jax.ShapeDtypeStruct((B, NBLK, C, K_NB, R), jnp.float32),
            jax.ShapeDtypeStruct((C, R), jnp.float32),
            jax.ShapeDtypeStruct((C, R), jnp.float32),
        ],
    )(h1, a1[None, :], c1[None, :], W2, b2[None, :])

    mean2 = jnp.sum(s2, axis=1) / cnt
    var2 = jnp.sum(q2, axis=1) / cnt - mean2 * mean2
    a2 = g2 / jnp.sqrt(var2 + 1e-5)
    c2 = be2 - mean2 * a2

    out = pl.pallas_call(
        _k4,
        grid=(B, NBLK),
        in_specs=[
            pl.BlockSpec((1, 1, C, K_NB, R), lambda b, i: (b, i, 0, 0, 0)),
            pl.BlockSpec((1, C), lambda b, i: (0, 0)),
            pl.BlockSpec((1, C), lambda b, i: (0, 0)),
            pl.BlockSpec((C, C), lambda b, i: (0, 0)),
            pl.BlockSpec((1, C), lambda b, i: (0, 0)),
        ],
        out_specs=[
            pl.BlockSpec((1, C, 1, 1, R), lambda b, i: (b, 0, i, 0, 0)),
        ],
        out_shape=[
            jax.ShapeDtypeStruct((B, C, NBLK, 1, R), jnp.float32),
        ],
    )(h2, a2[None, :], c2[None, :], W3, b3[None, :])[0]

    return out.reshape(B, C, N)


# speedup vs baseline: 826.6682x; 1.2758x over previous
import jax
import jax.numpy as jnp
from jax.experimental import pallas as pl

K_NB = 8
C = 12
R = 512
NBLK = 4


def _jacobi_eigh(a00, a11, a22, a01, a02, a12):
    one = jnp.float32(1.0)
    zero = jnp.float32(0.0)
    A = {(0, 0): a00, (1, 1): a11, (2, 2): a22,
         (0, 1): a01, (0, 2): a02, (1, 2): a12}
    V = {}
    for i in range(3):
        for j in range(3):
            V[(i, j)] = jnp.full_like(a00, one if i == j else zero)

    def getA(i, j):
        return A[(i, j)] if i <= j else A[(j, i)]

    def setA(i, j, v):
        A[(i, j) if i <= j else (j, i)] = v

    for _ in range(15):
        for (p, q) in ((0, 2), (1, 2), (0, 1)):
            r = 3 - p - q
            apq = getA(p, q)
            app = getA(p, p)
            aqq = getA(q, q)
            tau = (aqq - app) / (2.0 * apq)
            t = jnp.sign(tau) / (jnp.abs(tau) + jnp.sqrt(1.0 + tau * tau))
            t = jnp.where(tau == 0.0, one, t)
            small = jnp.abs(apq) <= 1e-6 * jnp.sqrt(jnp.abs(app * aqq))
            t = jnp.where(small | (apq == 0.0), zero, t)
            c = 1.0 / jnp.sqrt(1.0 + t * t)
            s = t * c
            apr = getA(p, r)
            aqr = getA(q, r)
            b_pp = c * app - s * apq
            b_pq = s * app + c * apq
            b_qp = c * apq - s * aqq
            b_qq = s * apq + c * aqq
            b_rp = c * apr - s * aqr
            b_rq = s * apr + c * aqr
            setA(p, p, c * b_pp - s * b_qp)
            setA(q, q, s * b_pq + c * b_qq)
            setA(p, q, c * b_pq - s * b_qq)
            setA(p, r, b_rp)
            setA(q, r, b_rq)
            for i in range(3):
                vip = V[(i, p)]
                viq = V[(i, q)]
                V[(i, p)] = c * vip - s * viq
                V[(i, q)] = s * vip + c * viq

    lam = [A[(0, 0)], A[(1, 1)], A[(2, 2)]]
    cols = [[V[(i, 0)] for i in range(3)],
            [V[(i, 1)] for i in range(3)],
            [V[(i, 2)] for i in range(3)]]

    def cswap(cond, x, y):
        return jnp.where(cond, y, x), jnp.where(cond, x, y)

    for (i, j) in ((0, 1), (1, 2), (0, 1)):
        sw = lam[j] < lam[i]
        lam[i], lam[j] = cswap(sw, lam[i], lam[j])
        for d in range(3):
            cols[i][d], cols[j][d] = cswap(sw, cols[i][d], cols[j][d])
    return lam, cols


def _k1(cb_ref, cat_ref, w1_ref, h1_ref, s1_ref, q1_ref):
    cb = cb_ref[0]
    caT = cat_ref[0]
    N = caT.shape[1]
    cax = caT[0:1, :]
    cay = caT[1:2, :]
    caz = caT[2:3, :]
    d2b = cb[:, 0] ** 2 + cb[:, 1] ** 2 + cb[:, 2] ** 2
    d2a = cax * cax + cay * cay + caz * caz
    prod = jnp.dot(cb, caT, preferred_element_type=jnp.float32)
    d = d2b[:, None] + d2a - 2.0 * prod

    iota = jax.lax.broadcasted_iota(jnp.int32, (R, N), 1).astype(jnp.float32)
    idxs = []
    for _ in range(K_NB):
        m = jnp.min(d, axis=1)
        cand = jnp.where(d == m[:, None], iota, jnp.float32(N))
        idx = jnp.min(cand, axis=1)
        idxs.append(idx)
        d = jnp.where(iota == idx[:, None], jnp.inf, d)

    idx8 = jnp.stack(idxs, axis=0).astype(jnp.int32)
    ch = idx8 // 128
    li = idx8 - ch * 128

    def gather_row(vec_row):
        out = jnp.zeros((K_NB, R), jnp.float32)
        for k in range(N // 128):
            src = jnp.broadcast_to(vec_row[:, k * 128:(k + 1) * 128],
                                   (K_NB, 128))
            val = jnp.take_along_axis(src, li, axis=1)
            out = jnp.where(ch == k, val, out)
        return out

    gx = gather_row(cax)
    gy = gather_row(cay)
    gz = gather_row(caz)

    def bf(x):
        return x.astype(jnp.bfloat16).astype(jnp.float32)

    def cov_entry(u, v):
        p = [u[j] * v[j] for j in range(K_NB)]
        return ((p[0] + p[1]) + (p[2] + p[3])) + ((p[4] + p[5]) + (p[6] + p[7]))

    dxs = [bf(gx[j:j + 1, :] - gx) for j in range(K_NB)]
    dys = [bf(gy[j:j + 1, :] - gy) for j in range(K_NB)]
    dzs = [bf(gz[j:j + 1, :] - gz) for j in range(K_NB)]
    c00 = cov_entry(dxs, dxs)
    c11 = cov_entry(dys, dys)
    c22 = cov_entry(dzs, dzs)
    c01 = cov_entry(dxs, dys)
    c02 = cov_entry(dxs, dzs)
    c12 = cov_entry(dys, dzs)

    lam, cols = _jacobi_eigh(c00, c11, c22, c01, c02, c12)
    l3, l2, l1 = lam[0], lam[1], lam[2]
    v1 = cols[2]
    v2 = cols[1]
    v3 = [-cols[0][i] for i in range(3)]
    n1 = jnp.sqrt(v1[0] ** 2 + v1[1] ** 2 + v1[2] ** 2)
    n2 = jnp.sqrt(v2[0] ** 2 + v2[1] ** 2 + v2[2] ** 2)
    n3 = jnp.sqrt(v3[0] ** 2 + v3[1] ** 2 + v3[2] ** 2)
    l1 = l1 / n1
    l2 = l2 / n2
    l3 = l3 / n3
    v1 = [v1[i] / n1 for i in range(3)]
    v2 = [v2[i] / n2 for i in range(3)]
    v3 = [v3[i] / n3 for i in range(3)]
    fa = (l1 - l2) / l1
    fp = (l2 - l3) / l1
    fs = l3 / l1
    feats = [fa, fp, fs, v1[0], v1[1], v1[2], v2[0], v2[1], v2[2],
             v3[0], v3[1], v3[2]]

    first = (pl.program_id(0) == 0) & (pl.program_id(1) == 0)

    @pl.when(first)
    def _():
        s1_ref[...] = jnp.zeros_like(s1_ref)
        q1_ref[...] = jnp.zeros_like(q1_ref)

    W1 = w1_ref[...]
    for o in range(C):
        acc = W1[o:o + 1, 0:1] * feats[0]
        for cc in range(1, C):
            acc = acc + W1[o:o + 1, cc:cc + 1] * feats[cc]
        h1_ref[0, 0, o] = acc
        s1_ref[o, :] = s1_ref[o, :] + jnp.sum(acc, axis=0)
        q1_ref[o, :] = q1_ref[o, :] + jnp.sum(acc * acc, axis=0)


def _k3(h1_ref, a1_ref, c1_ref, w2_ref, b2_ref, h2_ref, s2_ref, q2_ref):
    first = (pl.program_id(0) == 0) & (pl.program_id(1) == 0)

    @pl.when(first)
    def _():
        s2_ref[...] = jnp.zeros_like(s2_ref)
        q2_ref[...] = jnp.zeros_like(q2_ref)

    a1 = a1_ref[...]
    c1 = c1_ref[...]
    W2 = w2_ref[...]
    b2 = b2_ref[...]
    xs = []
    for cc in range(C):
        xs.append(jax.nn.relu(a1[0:1, cc:cc + 1] * h1_ref[0, 0, cc]
                              + c1[0:1, cc:cc + 1]))
    for o in range(C):
        acc = W2[o:o + 1, 0:1] * xs[0]
        for cc in range(1, C):
            acc = acc + W2[o:o + 1, cc:cc + 1] * xs[cc]
        acc = acc + b2[0:1, o:o + 1]
        h2_ref[0, 0, o] = acc
        s2_ref[o, :] = s2_ref[o, :] + jnp.sum(acc, axis=0)
        q2_ref[o, :] = q2_ref[o, :] + jnp.sum(acc * acc, axis=0)


def _k4(h2_ref, a2_ref, c2_ref, w3_ref, b3_ref, o_ref):
    a2 = a2_ref[...]
    c2 = c2_ref[...]
    W3 = w3_ref[...]
    b3 = b3_ref[...]
    xs = []
    for cc in range(C):
        xs.append(jax.nn.relu(a2[0:1, cc:cc + 1] * h2_ref[0, 0, cc]
                              + c2[0:1, cc:cc + 1]))
    for o in range(C):
        acc = W3[o:o + 1, 0:1] * xs[0]
        for cc in range(1, C):
            acc = acc + W3[o:o + 1, cc:cc + 1] * xs[cc]
        acc = acc + b3[0:1, o:o + 1]
        o_ref[0, o, 0, 0] = jnp.sum(acc, axis=0)


def kernel(center, W1, g1, be1, W2, b2, g2, be2, W3, b3):
    B, N, _ = center.shape
    cnt = jnp.float32(B * N * K_NB)

    centerT = jnp.transpose(center, (0, 2, 1))
    h1, s1, q1 = pl.pallas_call(
        _k1,
        grid=(B, NBLK),
        in_specs=[
            pl.BlockSpec((1, R, 3), lambda b, i: (b, i, 0)),
            pl.BlockSpec((1, 3, N), lambda b, i: (b, 0, 0)),
            pl.BlockSpec((C, C), lambda b, i: (0, 0)),
        ],
        out_specs=[
            pl.BlockSpec((1, 1, C, K_NB, R), lambda b, i: (b, i, 0, 0, 0)),
            pl.BlockSpec((C, R), lambda b, i: (0, 0)),
            pl.BlockSpec((C, R), lambda b, i: (0, 0)),
        ],
        out_shape=[
            jax.ShapeDtypeStruct((B, NBLK, C, K_NB, R), jnp.float32),
            jax.ShapeDtypeStruct((C, R), jnp.float32),
            jax.ShapeDtypeStruct((C, R), jnp.float32),
        ],
    )(center, centerT, W1)

    mean1 = jnp.sum(s1, axis=1) / cnt
    var1 = jnp.sum(q1, axis=1) / cnt - mean1 * mean1
    a1 = g1 / jnp.sqrt(var1 + 1e-5)
    c1 = be1 - mean1 * a1

    h2, s2, q2 = pl.pallas_call(
        _k3,
        grid=(B, NBLK),
        in_specs=[
            pl.BlockSpec((1, 1, C, K_NB, R), lambda b, i: (b, i, 0, 0, 0)),
            pl.BlockSpec((1, C), lambda b, i: (0, 0)),
            pl.BlockSpec((1, C), lambda b, i: (0, 0)),
            pl.BlockSpec((C, C), lambda b, i: (0, 0)),
            pl.BlockSpec((1, C), lambda b, i: (0, 0)),
        ],
        out_specs=[
            pl.BlockSpec((1, 1, C, K_NB, R), lambda b, i: (b, i, 0, 0, 0)),
            pl.BlockSpec((C, R), lambda b, i: (0, 0)),
            pl.BlockSpec((C, R), lambda b, i: (0, 0)),
        ],
        out_shape=[
            jax.ShapeDtypeStruct((B, NBLK, C, K_NB, R), jnp.float32),
            jax.ShapeDtypeStruct((C, R), jnp.float32),
            jax.ShapeDtypeStruct((C, R), jnp.float32),
        ],
    )(h1, a1[None, :], c1[None, :], W2, b2[None, :])

    mean2 = jnp.sum(s2, axis=1) / cnt
    var2 = jnp.sum(q2, axis=1) / cnt - mean2 * mean2
    a2 = g2 / jnp.sqrt(var2 + 1e-5)
    c2 = be2 - mean2 * a2

    out = pl.pallas_call(
        _k4,
        grid=(B, NBLK),
        in_specs=[
            pl.BlockSpec((1, 1, C, K_NB, R), lambda b, i: (b, i, 0, 0, 0)),
            pl.BlockSpec((1, C), lambda b, i: (0, 0)),
            pl.BlockSpec((1, C), lambda b, i: (0, 0)),
            pl.BlockSpec((C, C), lambda b, i: (0, 0)),
            pl.BlockSpec((1, C), lambda b, i: (0, 0)),
        ],
        out_specs=[
            pl.BlockSpec((1, C, 1, 1, R), lambda b, i: (b, 0, i, 0, 0)),
        ],
        out_shape=[
            jax.ShapeDtypeStruct((B, C, NBLK, 1, R), jnp.float32),
        ],
    )(h2, a2[None, :], c2[None, :], W3, b3[None, :])[0]

    return out.reshape(B, C, N)


# speedup vs baseline: 896.0097x; 1.0839x over previous
import jax
import jax.numpy as jnp
from jax.experimental import pallas as pl

K_NB = 8
C = 12
R = 1024
NBLK = 2


def _jacobi_eigh(a00, a11, a22, a01, a02, a12):
    one = jnp.float32(1.0)
    zero = jnp.float32(0.0)
    A = {(0, 0): a00, (1, 1): a11, (2, 2): a22,
         (0, 1): a01, (0, 2): a02, (1, 2): a12}
    V = {}
    for i in range(3):
        for j in range(3):
            V[(i, j)] = jnp.full_like(a00, one if i == j else zero)

    def getA(i, j):
        return A[(i, j)] if i <= j else A[(j, i)]

    def setA(i, j, v):
        A[(i, j) if i <= j else (j, i)] = v

    for _ in range(15):
        for (p, q) in ((0, 2), (1, 2), (0, 1)):
            r = 3 - p - q
            apq = getA(p, q)
            app = getA(p, p)
            aqq = getA(q, q)
            tau = (aqq - app) / (2.0 * apq)
            t = jnp.sign(tau) / (jnp.abs(tau) + jnp.sqrt(1.0 + tau * tau))
            t = jnp.where(tau == 0.0, one, t)
            small = jnp.abs(apq) <= 1e-6 * jnp.sqrt(jnp.abs(app * aqq))
            t = jnp.where(small | (apq == 0.0), zero, t)
            c = 1.0 / jnp.sqrt(1.0 + t * t)
            s = t * c
            apr = getA(p, r)
            aqr = getA(q, r)
            b_pp = c * app - s * apq
            b_pq = s * app + c * apq
            b_qp = c * apq - s * aqq
            b_qq = s * apq + c * aqq
            b_rp = c * apr - s * aqr
            b_rq = s * apr + c * aqr
            setA(p, p, c * b_pp - s * b_qp)
            setA(q, q, s * b_pq + c * b_qq)
            setA(p, q, c * b_pq - s * b_qq)
            setA(p, r, b_rp)
            setA(q, r, b_rq)
            for i in range(3):
                vip = V[(i, p)]
                viq = V[(i, q)]
                V[(i, p)] = c * vip - s * viq
                V[(i, q)] = s * vip + c * viq

    lam = [A[(0, 0)], A[(1, 1)], A[(2, 2)]]
    cols = [[V[(i, 0)] for i in range(3)],
            [V[(i, 1)] for i in range(3)],
            [V[(i, 2)] for i in range(3)]]

    def cswap(cond, x, y):
        return jnp.where(cond, y, x), jnp.where(cond, x, y)

    for (i, j) in ((0, 1), (1, 2), (0, 1)):
        sw = lam[j] < lam[i]
        lam[i], lam[j] = cswap(sw, lam[i], lam[j])
        for d in range(3):
            cols[i][d], cols[j][d] = cswap(sw, cols[i][d], cols[j][d])
    return lam, cols


def _k1(cb_ref, cat_ref, w1_ref, h1_ref, s1_ref, q1_ref):
    cb = cb_ref[0]
    caT = cat_ref[0]
    N = caT.shape[1]
    cax = caT[0:1, :]
    cay = caT[1:2, :]
    caz = caT[2:3, :]
    d2b = cb[:, 0] ** 2 + cb[:, 1] ** 2 + cb[:, 2] ** 2
    d2a = cax * cax + cay * cay + caz * caz
    prod = jnp.dot(cb, caT, preferred_element_type=jnp.float32)
    d = d2b[:, None] + d2a - 2.0 * prod

    iota = jax.lax.broadcasted_iota(jnp.int32, (R, N), 1).astype(jnp.float32)
    idxs = []
    for _ in range(K_NB):
        m = jnp.min(d, axis=1)
        cand = jnp.where(d == m[:, None], iota, jnp.float32(N))
        idx = jnp.min(cand, axis=1)
        idxs.append(idx)
        d = jnp.where(iota == idx[:, None], jnp.inf, d)

    idx8 = jnp.stack(idxs, axis=0).astype(jnp.int32)
    ch = idx8 // 128
    li = idx8 - ch * 128

    def gather_row(vec_row):
        out = jnp.zeros((K_NB, R), jnp.float32)
        for k in range(N // 128):
            src = jnp.broadcast_to(vec_row[:, k * 128:(k + 1) * 128],
                                   (K_NB, 128))
            val = jnp.take_along_axis(src, li, axis=1)
            out = jnp.where(ch == k, val, out)
        return out

    gx = gather_row(cax)
    gy = gather_row(cay)
    gz = gather_row(caz)

    def bf(x):
        return x.astype(jnp.bfloat16).astype(jnp.float32)

    def cov_entry(u, v):
        p = [u[j] * v[j] for j in range(K_NB)]
        return ((p[0] + p[1]) + (p[2] + p[3])) + ((p[4] + p[5]) + (p[6] + p[7]))

    dxs = [bf(gx[j:j + 1, :] - gx) for j in range(K_NB)]
    dys = [bf(gy[j:j + 1, :] - gy) for j in range(K_NB)]
    dzs = [bf(gz[j:j + 1, :] - gz) for j in range(K_NB)]
    c00 = cov_entry(dxs, dxs)
    c11 = cov_entry(dys, dys)
    c22 = cov_entry(dzs, dzs)
    c01 = cov_entry(dxs, dys)
    c02 = cov_entry(dxs, dzs)
    c12 = cov_entry(dys, dzs)

    lam, cols = _jacobi_eigh(c00, c11, c22, c01, c02, c12)
    l3, l2, l1 = lam[0], lam[1], lam[2]
    v1 = cols[2]
    v2 = cols[1]
    v3 = [-cols[0][i] for i in range(3)]
    n1 = jnp.sqrt(v1[0] ** 2 + v1[1] ** 2 + v1[2] ** 2)
    n2 = jnp.sqrt(v2[0] ** 2 + v2[1] ** 2 + v2[2] ** 2)
    n3 = jnp.sqrt(v3[0] ** 2 + v3[1] ** 2 + v3[2] ** 2)
    l1 = l1 / n1
    l2 = l2 / n2
    l3 = l3 / n3
    v1 = [v1[i] / n1 for i in range(3)]
    v2 = [v2[i] / n2 for i in range(3)]
    v3 = [v3[i] / n3 for i in range(3)]
    fa = (l1 - l2) / l1
    fp = (l2 - l3) / l1
    fs = l3 / l1
    feats = [fa, fp, fs, v1[0], v1[1], v1[2], v2[0], v2[1], v2[2],
             v3[0], v3[1], v3[2]]

    first = (pl.program_id(0) == 0) & (pl.program_id(1) == 0)

    @pl.when(first)
    def _():
        s1_ref[...] = jnp.zeros_like(s1_ref)
        q1_ref[...] = jnp.zeros_like(q1_ref)

    W1 = w1_ref[...]
    for o in range(C):
        acc = W1[o:o + 1, 0:1] * feats[0]
        for cc in range(1, C):
            acc = acc + W1[o:o + 1, cc:cc + 1] * feats[cc]
        h1_ref[0, 0, o] = acc
        s1_ref[o, :] = s1_ref[o, :] + jnp.sum(acc, axis=0)
        q1_ref[o, :] = q1_ref[o, :] + jnp.sum(acc * acc, axis=0)


def _k3(h1_ref, a1_ref, c1_ref, w2_ref, b2_ref, h2_ref, s2_ref, q2_ref):
    first = (pl.program_id(0) == 0) & (pl.program_id(1) == 0)

    @pl.when(first)
    def _():
        s2_ref[...] = jnp.zeros_like(s2_ref)
        q2_ref[...] = jnp.zeros_like(q2_ref)

    a1 = a1_ref[...]
    c1 = c1_ref[...]
    W2 = w2_ref[...]
    b2 = b2_ref[...]
    xs = []
    for cc in range(C):
        xs.append(jax.nn.relu(a1[0:1, cc:cc + 1] * h1_ref[0, 0, cc]
                              + c1[0:1, cc:cc + 1]))
    for o in range(C):
        acc = W2[o:o + 1, 0:1] * xs[0]
        for cc in range(1, C):
            acc = acc + W2[o:o + 1, cc:cc + 1] * xs[cc]
        acc = acc + b2[0:1, o:o + 1]
        h2_ref[0, 0, o] = acc
        s2_ref[o, :] = s2_ref[o, :] + jnp.sum(acc, axis=0)
        q2_ref[o, :] = q2_ref[o, :] + jnp.sum(acc * acc, axis=0)


def _k4(h2_ref, a2_ref, c2_ref, w3_ref, b3_ref, o_ref):
    a2 = a2_ref[...]
    c2 = c2_ref[...]
    W3 = w3_ref[...]
    b3 = b3_ref[...]
    xs = []
    for cc in range(C):
        xs.append(jax.nn.relu(a2[0:1, cc:cc + 1] * h2_ref[0, 0, cc]
                              + c2[0:1, cc:cc + 1]))
    for o in range(C):
        acc = W3[o:o + 1, 0:1] * xs[0]
        for cc in range(1, C):
            acc = acc + W3[o:o + 1, cc:cc + 1] * xs[cc]
        acc = acc + b3[0:1, o:o + 1]
        o_ref[0, o, 0, 0] = jnp.sum(acc, axis=0)


def kernel(center, W1, g1, be1, W2, b2, g2, be2, W3, b3):
    B, N, _ = center.shape
    cnt = jnp.float32(B * N * K_NB)

    centerT = jnp.transpose(center, (0, 2, 1))
    h1, s1, q1 = pl.pallas_call(
        _k1,
        grid=(B, NBLK),
        in_specs=[
            pl.BlockSpec((1, R, 3), lambda b, i: (b, i, 0)),
            pl.BlockSpec((1, 3, N), lambda b, i: (b, 0, 0)),
            pl.BlockSpec((C, C), lambda b, i: (0, 0)),
        ],
        out_specs=[
            pl.BlockSpec((1, 1, C, K_NB, R), lambda b, i: (b, i, 0, 0, 0)),
            pl.BlockSpec((C, R), lambda b, i: (0, 0)),
            pl.BlockSpec((C, R), lambda b, i: (0, 0)),
        ],
        out_shape=[
            jax.ShapeDtypeStruct((B, NBLK, C, K_NB, R), jnp.float32),
            jax.ShapeDtypeStruct((C, R), jnp.float32),
            jax.ShapeDtypeStruct((C, R), jnp.float32),
        ],
    )(center, centerT, W1)

    mean1 = jnp.sum(s1, axis=1) / cnt
    var1 = jnp.sum(q1, axis=1) / cnt - mean1 * mean1
    a1 = g1 / jnp.sqrt(var1 + 1e-5)
    c1 = be1 - mean1 * a1

    h2, s2, q2 = pl.pallas_call(
        _k3,
        grid=(B, NBLK),
        in_specs=[
            pl.BlockSpec((1, 1, C, K_NB, R), lambda b, i: (b, i, 0, 0, 0)),
            pl.BlockSpec((1, C), lambda b, i: (0, 0)),
            pl.BlockSpec((1, C), lambda b, i: (0, 0)),
            pl.BlockSpec((C, C), lambda b, i: (0, 0)),
            pl.BlockSpec((1, C), lambda b, i: (0, 0)),
        ],
        out_specs=[
            pl.BlockSpec((1, 1, C, K_NB, R), lambda b, i: (b, i, 0, 0, 0)),
            pl.BlockSpec((C, R), lambda b, i: (0, 0)),
            pl.BlockSpec((C, R), lambda b, i: (0, 0)),
        ],
        out_shape=[
            jax.ShapeDtypeStruct((B, NBLK, C, K_NB, R), jnp.float32),
            jax.ShapeDtypeStruct((C, R), jnp.float32),
            jax.ShapeDtypeStruct((C, R), jnp.float32),
        ],
    )(h1, a1[None, :], c1[None, :], W2, b2[None, :])

    mean2 = jnp.sum(s2, axis=1) / cnt
    var2 = jnp.sum(q2, axis=1) / cnt - mean2 * mean2
    a2 = g2 / jnp.sqrt(var2 + 1e-5)
    c2 = be2 - mean2 * a2

    out = pl.pallas_call(
        _k4,
        grid=(B, NBLK),
        in_specs=[
            pl.BlockSpec((1, 1, C, K_NB, R), lambda b, i: (b, i, 0, 0, 0)),
            pl.BlockSpec((1, C), lambda b, i: (0, 0)),
            pl.BlockSpec((1, C), lambda b, i: (0, 0)),
            pl.BlockSpec((C, C), lambda b, i: (0, 0)),
            pl.BlockSpec((1, C), lambda b, i: (0, 0)),
        ],
        out_specs=[
            pl.BlockSpec((1, C, 1, 1, R), lambda b, i: (b, 0, i, 0, 0)),
        ],
        out_shape=[
            jax.ShapeDtypeStruct((B, C, NBLK, 1, R), jnp.float32),
        ],
    )(h2, a2[None, :], c2[None, :], W3, b3[None, :])[0]

    return out.reshape(B, C, N)
